# baseline (speedup 1.0000x reference)
import functools

import jax
import jax.numpy as jnp
from jax import lax
from jax.experimental import pallas as pl
from jax.experimental.pallas import tpu as pltpu
from jax.experimental.pallas import tpu_sc as plsc

NC = 2
NS = 16
LN = 16


def _sc_aggregate(h, src, dst, w):
  N, H = h.shape
  P, E = src.shape
  NW = NC * NS
  CH = 80
  NB = 4
  per_w_raw = -(-E // NW)
  NCH = -(-per_w_raw // CH)
  if NCH % NB:
    NCH += NB - NCH % NB
  per_w = NCH * CH
  K0 = int(2 * NCH / (1 + 2.8)) // NB * NB
  K1 = 2 * NCH - K0
  pad_total = NW * per_w - E

  def pad_edges(x, value):
    xp = jnp.pad(x, ((0, 0), (0, pad_total)), constant_values=value)
    return xp.reshape(P * NW * per_w)

  src4 = pad_edges(src, 0)
  dst4 = pad_edges(dst, 0)
  w4 = pad_edges(w, 0.0)

  RA = (N // NS) // 8 * 8
  RB = N - RA * (NS - 1)
  assert RB % 8 == 0 and RB <= 2 * RA

  mesh = plsc.VectorSubcoreMesh(core_axis_name="c", subcore_axis_name="s",
                                num_cores=NC, num_subcores=NS)

  def _scale_rows(rows_ref, w_ref):
    def body(g, carry):
      wv = w_ref[pl.ds(g * LN, LN)]
      for j in range(LN):
        wi = wv[j]
        row = g * LN + j
        for c in range(H // LN):
          sl = pl.ds(c * LN, LN)
          rows_ref[row, sl] = rows_ref[row, sl] * wi
      return carry
    lax.fori_loop(0, CH // LN, body, 0)

  @functools.partial(
      pl.kernel,
      out_type=jax.ShapeDtypeStruct((P, NC, N, H), jnp.float32),
      mesh=mesh,
      scratch_types=[
          pltpu.VMEM_SHARED((N, H), jnp.float32),
          [pltpu.VMEM((CH,), jnp.int32) for _ in range(NB)],
          [pltpu.VMEM((CH,), jnp.int32) for _ in range(NB)],
          [pltpu.VMEM((CH,), jnp.float32) for _ in range(NB)],
          pltpu.VMEM((NB, CH, H), jnp.float32),
          pltpu.SemaphoreType.DMA((NB,)),
          pltpu.SemaphoreType.DMA((NB,)),
          pltpu.SemaphoreType.DMA((NB,)),
      ],
  )
  def body(h_hbm, src_hbm, dst_hbm, w_hbm, zero_hbm, parts_hbm,
           acc, src_v, dst_v, w_v, rows, isem, gsem, ssem):
    cid = lax.axis_index("c")
    sid = lax.axis_index("s")
    wid = sid * NC + cid

    def idx_start(i, b):
      off = i * CH
      pltpu.async_copy(src_hbm.at[pl.ds(off, CH)], src_v[b], isem.at[b])
      pltpu.async_copy(dst_hbm.at[pl.ds(off, CH)], dst_v[b], isem.at[b])
      pltpu.async_copy(w_hbm.at[pl.ds(off, CH)], w_v[b], isem.at[b])

    def idx_wait(i, b):
      off = i * CH
      pltpu.make_async_copy(src_hbm.at[pl.ds(off, CH)], src_v[b],
                            isem.at[b]).wait()
      pltpu.make_async_copy(dst_hbm.at[pl.ds(off, CH)], dst_v[b],
                            isem.at[b]).wait()
      pltpu.make_async_copy(w_hbm.at[pl.ds(off, CH)], w_v[b],
                            isem.at[b]).wait()

    def gather_start(b):
      pltpu.async_copy(h_hbm.at[src_v[b]], rows.at[b], gsem.at[b])

    def gather_wait(b):
      pltpu.make_async_copy(h_hbm.at[src_v[b]], rows.at[b],
                            gsem.at[b]).wait()

    def scatter_start(b):
      pltpu.async_copy(rows.at[b], acc.at[dst_v[b]], ssem.at[b], add=True)

    def scatter_wait(b):
      pltpu.make_async_copy(rows.at[b], acc.at[dst_v[b]], ssem.at[b]).wait()

    for p in range(P):
      @pl.when(sid < NS - 1)
      def _():
        pltpu.sync_copy(zero_hbm.at[pl.ds(0, RA)],
                        acc.at[pl.ds(sid * RA, RA)])

      @pl.when(sid == NS - 1)
      def _():
        pltpu.sync_copy(zero_hbm, acc.at[pl.ds((NS - 1) * RA, RB)])

      plsc.subcore_barrier()
      g0 = jnp.where(cid == 0, sid * K0, NS * K0 + sid * K1)
      K = jnp.where(cid == 0, K0, K1)
      base = p * NW * NCH + g0

      idx_start(base + 0, 0)
      idx_start(base + 1, 1)
      idx_wait(base + 0, 0)
      gather_start(0)

      def quad(k, carry):
        for bb in range(NB):
          i = k * NB + bb
          b1 = (bb + 1) % NB
          b2 = (bb + 2) % NB
          gather_wait(bb)

          @pl.when(i >= 2)
          def _():
            scatter_wait(b2)

          @pl.when(i + 1 < K)
          def _():
            idx_wait(base + i + 1, b1)
            gather_start(b1)

          @pl.when(i + 2 < K)
          def _():
            idx_start(base + i + 2, b2)
          _scale_rows(rows.at[bb], w_v[bb])
          scatter_start(bb)
        return carry
      lax.fori_loop(0, K // NB, quad, 0)
      scatter_wait((NB - 2) % NB)
      scatter_wait((NB - 1) % NB)

      plsc.subcore_barrier()

      @pl.when(sid < NS - 1)
      def _():
        pltpu.sync_copy(
            acc.at[pl.ds(sid * RA, RA)],
            parts_hbm.at[p, cid, pl.ds(sid * RA, RA)])

      @pl.when(sid == NS - 1)
      def _():
        pltpu.sync_copy(
            acc.at[pl.ds((NS - 1) * RA, RB)],
            parts_hbm.at[p, cid, pl.ds((NS - 1) * RA, RB)])

      plsc.subcore_barrier()

  zeros_slab = jnp.zeros((RB, H), dtype=jnp.float32)
  return body(h, src4, dst4, w4, zeros_slab)


def _tc_transform(parts, W, b, prelu_a, fc_W, fc_b):
  P, N, H = parts.shape
  BN = 1000 if N % 1000 == 0 else N
  nb = N // BN

  def body(parts_ref, W_ref, b_ref, a_ref, fcW_ref, fcb_ref,
           emb_ref, sacc_ref):
    i = pl.program_id(0)

    @pl.when(i == 0)
    def _():
      sacc_ref[...] = jnp.zeros_like(sacc_ref)

    for p in range(P):
      agg = parts_ref[p]
      fts = lax.dot_general(agg, W_ref[p], (((1,), (1,)), ((), ())),
                            preferred_element_type=jnp.float32)
      x = fts + b_ref[p:p + 1, :]
      a = a_ref[0, p]
      e = jnp.where(x > 0, x, a * x)
      emb_ref[p] = e
      t = jnp.tanh(
          lax.dot_general(e, fcW_ref[...], (((1,), (1,)), ((), ())),
                          preferred_element_type=jnp.float32)
          + fcb_ref[...])
      sacc_ref[p:p + 1, :] += jnp.sum(t, axis=0, keepdims=True)

  emb, sacc = pl.pallas_call(
      body,
      grid=(nb,),
      in_specs=[
          pl.BlockSpec((P, BN, H), lambda i: (0, i, 0)),
          pl.BlockSpec((P, H, H), lambda i: (0, 0, 0)),
          pl.BlockSpec((P, H), lambda i: (0, 0)),
          pl.BlockSpec(memory_space=pltpu.SMEM),
          pl.BlockSpec((H, H), lambda i: (0, 0)),
          pl.BlockSpec((1, H), lambda i: (0, 0)),
      ],
      out_specs=[
          pl.BlockSpec((P, BN, H), lambda i: (0, i, 0)),
          pl.BlockSpec((P, H), lambda i: (0, 0)),
      ],
      out_shape=[
          jax.ShapeDtypeStruct((P, N, H), jnp.float32),
          jax.ShapeDtypeStruct((P, H), jnp.float32),
      ],
  )(parts, W, b, prelu_a.reshape(1, P), fc_W, fc_b.reshape(1, H))
  return emb, sacc


def _tc_combine(emb, sacc, att, n_nodes):
  P, N, H = emb.shape
  BN = 1000 if N % 1000 == 0 else N
  nb = N // BN

  def body(emb_ref, sacc_ref, att_ref, z_ref):
    logits = [
        jnp.sum(att_ref[...] * sacc_ref[p:p + 1, :], axis=1, keepdims=True)
        / n_nodes
        for p in range(P)
    ]
    m = logits[0]
    for p in range(1, P):
      m = jnp.maximum(m, logits[p])
    exps = [jnp.exp(l - m) for l in logits]
    se = exps[0]
    for p in range(1, P):
      se = se + exps[p]
    acc = (exps[0] / se) * emb_ref[0]
    for p in range(1, P):
      acc = acc + (exps[p] / se) * emb_ref[p]
    z_ref[...] = acc

  return pl.pallas_call(
      body,
      grid=(nb,),
      in_specs=[
          pl.BlockSpec((P, BN, H), lambda i: (0, i, 0)),
          pl.BlockSpec((P, H), lambda i: (0, 0)),
          pl.BlockSpec((1, H), lambda i: (0, 0)),
      ],
      out_specs=pl.BlockSpec((BN, H), lambda i: (i, 0)),
      out_shape=jax.ShapeDtypeStruct((N, H), jnp.float32),
  )(emb, sacc, att.reshape(1, H))


def _sc_aggregate_cols(h, src, dst, w):
  N, H = h.shape
  P, E = src.shape
  NW = NC * NS
  CW = H // NW
  SB = max(int(N - 1).bit_length(), 1)
  assert (N << SB) < 2**31
  CHE = 2560
  nch = -(-E // CHE)
  Ep = nch * CHE
  pad = Ep - E

  def pad_flat(x, value):
    return jnp.pad(x, ((0, 0), (0, pad)),
                   constant_values=value).reshape(P * Ep)

  packed = pad_flat((dst << SB) | src, 0)
  wflat = pad_flat(w, 0.0)
  hcols = h.reshape(N, NW, CW).transpose(1, 0, 2).reshape(NW, N * CW)

  mesh = plsc.VectorSubcoreMesh(core_axis_name="c", subcore_axis_name="s",
                                num_cores=NC, num_subcores=NS)
  NBUF = 2

  @functools.partial(
      pl.kernel,
      out_type=jax.ShapeDtypeStruct((P, NW, N * CW), jnp.float32),
      mesh=mesh,
      scratch_types=[
          pltpu.VMEM((N * CW,), jnp.float32),
          pltpu.VMEM((N * CW,), jnp.float32),
          [pltpu.VMEM((CHE,), jnp.int32) for _ in range(NBUF)],
          [pltpu.VMEM((CHE,), jnp.float32) for _ in range(NBUF)],
          pltpu.SemaphoreType.DMA((NBUF,)),
      ],
      compiler_params=pltpu.CompilerParams(needs_layout_passes=False),
  )
  def body(h_hbm, pk_hbm, w_hbm, parts_hbm, h_t, acc_t, pk_v, w_v, isem):
    cid = lax.axis_index("c")
    sid = lax.axis_index("s")
    wid = sid * NC + cid
    pltpu.sync_copy(h_hbm.at[wid], h_t)
    zero16 = jnp.zeros((LN,), jnp.float32)
    mask_src = jnp.full((LN,), (1 << SB) - 1, jnp.int32)


    def chunk_start(ci, bb):
      off = ci * CHE
      pltpu.async_copy(pk_hbm.at[pl.ds(off, CHE)], pk_v[bb], isem.at[bb])
      pltpu.async_copy(w_hbm.at[pl.ds(off, CHE)], w_v[bb], isem.at[bb])

    def chunk_wait(ci, bb):
      off = ci * CHE
      pltpu.make_async_copy(pk_hbm.at[pl.ds(off, CHE)], pk_v[bb],
                            isem.at[bb]).wait()
      pltpu.make_async_copy(w_hbm.at[pl.ds(off, CHE)], w_v[bb],
                            isem.at[bb]).wait()

    for p in range(P):
      base = p * nch

      def zero(i, carry):
        acc_t[pl.ds(i * LN, LN)] = zero16
        return carry
      lax.fori_loop(0, N * CW // LN, zero, 0)

      def cix(i):
        return base + i

      chunk_start(cix(0), 0)
      chunk_start(cix(1), 1)

      def process(i, bb):
        chunk_wait(cix(i), bb)

        def vbody(v, carry):
          sl = pl.ds(v * LN, LN)
          pk16 = pk_v[bb][sl]
          wv16 = w_v[bb][sl]
          si = (pk16 & mask_src) * CW
          di = lax.shift_right_logical(pk16, SB) * CW
          vals = [plsc.load_gather(h_t, [si + c]) * wv16 for c in range(CW)]
          cnt, _ = plsc.scan_count(di)
          rmin = jnp.min(cnt)
          rmax = jnp.max(cnt)

          def rbody(r, carry2):
            m = cnt == r
            for c in range(CW):
              plsc.addupdate_scatter(acc_t, [di + c], vals[c], mask=m)
            return carry2
          lax.fori_loop(rmin, rmax + 1, rbody, 0)
          return carry
        lax.fori_loop(0, CHE // LN, vbody, 0, unroll=2)

        @pl.when(i + NBUF < nch)
        def _():
          chunk_start(cix(i + NBUF), bb)

      def duo(k, carry):
        for bb in range(NBUF):
          i = k * NBUF + bb

          @pl.when(i < nch)
          def _():
            process(i, bb)
        return carry
      lax.fori_loop(0, -(-nch // NBUF), duo, 0)

      pltpu.sync_copy(acc_t, parts_hbm.at[p, wid])

  return body(hcols, packed, wflat)


def kernel(h, edge_index, edge_weight, W, b, prelu_a, fc_W, fc_b, att):
  N, H = h.shape
  P = edge_index.shape[0]
  NW = NC * NS
  CW = H // NW
  dst = edge_index[:, 0, :]
  src = edge_index[:, 1, :]
  parts = _sc_aggregate_cols(h, src, dst, edge_weight)
  agg = parts.reshape(P, NW, N, CW).transpose(0, 2, 1, 3).reshape(P, N, H)
  emb, sacc = _tc_transform(agg, W, b, prelu_a, fc_W, fc_b)
  return _tc_combine(emb, sacc, att, float(N))


# speedup vs baseline: 4.3585x; 4.3585x over previous
import functools

import jax
import jax.numpy as jnp
from jax import lax
from jax.experimental import pallas as pl
from jax.experimental.pallas import tpu as pltpu
from jax.experimental.pallas import tpu_sc as plsc

NC = 2
NS = 16
LN = 16


def _sc_aggregate(h, src, dst, w):
  N, H = h.shape
  P, E = src.shape
  src = src.reshape(P * E)
  dst = dst.reshape(P * E)
  w = w.reshape(P * E)
  NW = NC * NS
  per_w = E // NW
  CH = 128
  n_chunks = per_w // CH
  tail = per_w - n_chunks * CH
  RA = (N // NS) // 8 * 8
  RB = N - RA * (NS - 1)
  assert RB % 8 == 0 and RB <= 2 * RA
  mesh = plsc.VectorSubcoreMesh(core_axis_name="c", subcore_axis_name="s",
                                num_cores=NC, num_subcores=NS)

  def _scale_rows(rows_ref, w_ref, k):
    def body(g, carry):
      wv = w_ref[pl.ds(g * LN, LN)]
      for j in range(LN):
        wi = wv[j]
        row = g * LN + j
        for c in range(H // LN):
          sl = pl.ds(c * LN, LN)
          rows_ref[row, sl] = rows_ref[row, sl] * wi
      return carry
    lax.fori_loop(0, k // LN, body, 0)

  @functools.partial(
      pl.kernel,
      out_type=jax.ShapeDtypeStruct((P, NC, N, H), jnp.float32),
      mesh=mesh,
      scratch_types=[
          pltpu.VMEM_SHARED((N, H), jnp.float32),
          [pltpu.VMEM((CH,), jnp.int32) for _ in range(2)],
          [pltpu.VMEM((CH,), jnp.int32) for _ in range(2)],
          [pltpu.VMEM((CH,), jnp.float32) for _ in range(2)],
          pltpu.VMEM((CH, H), jnp.float32),
          pltpu.VMEM((LN,), jnp.int32),
          pltpu.VMEM((LN,), jnp.int32),
          pltpu.VMEM((LN,), jnp.float32),
          pltpu.VMEM((LN, H), jnp.float32),
          pltpu.SemaphoreType.DMA((2,)),
          pltpu.SemaphoreType.DMA,
      ],
  )
  def body(h_hbm, src_hbm, dst_hbm, w_hbm, zero_hbm, parts_hbm,
           acc, src_v, dst_v, w_v, rows_v, src_t, dst_t, w_t, rows_t,
           isem, sem):
    cid = lax.axis_index("c")
    sid = lax.axis_index("s")
    wid = sid * NC + cid

    def idx_start(off, b):
      pltpu.async_copy(src_hbm.at[pl.ds(off, CH)], src_v[b], isem.at[b])
      pltpu.async_copy(dst_hbm.at[pl.ds(off, CH)], dst_v[b], isem.at[b])
      pltpu.async_copy(w_hbm.at[pl.ds(off, CH)], w_v[b], isem.at[b])

    def idx_wait(off, b):
      pltpu.make_async_copy(src_hbm.at[pl.ds(off, CH)], src_v[b],
                            isem.at[b]).wait()
      pltpu.make_async_copy(dst_hbm.at[pl.ds(off, CH)], dst_v[b],
                            isem.at[b]).wait()
      pltpu.make_async_copy(w_hbm.at[pl.ds(off, CH)], w_v[b],
                            isem.at[b]).wait()

    for p in range(P):
      base = p * E + wid * per_w
      @pl.when(sid < NS - 1)
      def _():
        pltpu.sync_copy(zero_hbm.at[pl.ds(0, RA)],
                        acc.at[pl.ds(sid * RA, RA)])

      @pl.when(sid == NS - 1)
      def _():
        pltpu.sync_copy(zero_hbm, acc.at[pl.ds((NS - 1) * RA, RB)])

      plsc.subcore_barrier()
      idx_start(base, 0)

      assert n_chunks % 2 == 0

      def chunk(k, carry):
        for bb in range(2):
          i = k * 2 + bb
          off = base + i * CH
          idx_wait(off, bb)

          @pl.when(i + 1 < n_chunks)
          def _():
            idx_start(off + CH, 1 - bb)
          pltpu.async_copy(h_hbm.at[src_v[bb]], rows_v, sem).wait()
          _scale_rows(rows_v, w_v[bb], CH)
          pltpu.sync_copy(rows_v, acc.at[dst_v[bb]], add=True)
        return carry
      lax.fori_loop(0, n_chunks // 2, chunk, 0)

      if tail:
        off = base + n_chunks * CH
        pltpu.sync_copy(src_hbm.at[pl.ds(off, tail)], src_t)
        pltpu.sync_copy(dst_hbm.at[pl.ds(off, tail)], dst_t)
        pltpu.sync_copy(w_hbm.at[pl.ds(off, tail)], w_t)
        pltpu.async_copy(h_hbm.at[src_t], rows_t, sem).wait()
        _scale_rows(rows_t, w_t, tail)
        pltpu.sync_copy(rows_t, acc.at[dst_t], add=True)

      plsc.subcore_barrier()

      @pl.when(sid < NS - 1)
      def _():
        pltpu.sync_copy(
            acc.at[pl.ds(sid * RA, RA)],
            parts_hbm.at[p, cid, pl.ds(sid * RA, RA)])

      @pl.when(sid == NS - 1)
      def _():
        pltpu.sync_copy(
            acc.at[pl.ds((NS - 1) * RA, RB)],
            parts_hbm.at[p, cid, pl.ds((NS - 1) * RA, RB)])

      plsc.subcore_barrier()

  zeros_slab = jnp.zeros((RB, H), dtype=jnp.float32)
  return body(h, src, dst, w, zeros_slab)


def _tc_transform(parts, W, b, prelu_a, fc_W, fc_b):
  P, _, N, H = parts.shape
  BN = 1000 if N % 1000 == 0 else N
  nb = N // BN

  def body(parts_ref, W_ref, b_ref, a_ref, fcW_ref, fcb_ref,
           emb_ref, sacc_ref):
    i = pl.program_id(0)

    @pl.when(i == 0)
    def _():
      sacc_ref[...] = jnp.zeros_like(sacc_ref)

    for p in range(P):
      agg = parts_ref[p, 0] + parts_ref[p, 1]
      fts = lax.dot_general(agg, W_ref[p], (((1,), (1,)), ((), ())),
                            preferred_element_type=jnp.float32)
      x = fts + b_ref[p:p + 1, :]
      a = a_ref[0, p]
      e = jnp.where(x > 0, x, a * x)
      emb_ref[p] = e
      t = jnp.tanh(
          lax.dot_general(e, fcW_ref[...], (((1,), (1,)), ((), ())),
                          preferred_element_type=jnp.float32)
          + fcb_ref[...])
      sacc_ref[p:p + 1, :] += jnp.sum(t, axis=0, keepdims=True)

  emb, sacc = pl.pallas_call(
      body,
      grid=(nb,),
      in_specs=[
          pl.BlockSpec((P, 2, BN, H), lambda i: (0, 0, i, 0)),
          pl.BlockSpec((P, H, H), lambda i: (0, 0, 0)),
          pl.BlockSpec((P, H), lambda i: (0, 0)),
          pl.BlockSpec(memory_space=pltpu.SMEM),
          pl.BlockSpec((H, H), lambda i: (0, 0)),
          pl.BlockSpec((1, H), lambda i: (0, 0)),
      ],
      out_specs=[
          pl.BlockSpec((P, BN, H), lambda i: (0, i, 0)),
          pl.BlockSpec((P, H), lambda i: (0, 0)),
      ],
      out_shape=[
          jax.ShapeDtypeStruct((P, N, H), jnp.float32),
          jax.ShapeDtypeStruct((P, H), jnp.float32),
      ],
  )(parts, W, b, prelu_a.reshape(1, P), fc_W, fc_b.reshape(1, H))
  return emb, sacc


def _tc_combine(emb, sacc, att, n_nodes):
  P, N, H = emb.shape
  BN = 1000 if N % 1000 == 0 else N
  nb = N // BN

  def body(emb_ref, sacc_ref, att_ref, z_ref):
    logits = [
        jnp.sum(att_ref[...] * sacc_ref[p:p + 1, :], axis=1, keepdims=True)
        / n_nodes
        for p in range(P)
    ]
    m = logits[0]
    for p in range(1, P):
      m = jnp.maximum(m, logits[p])
    exps = [jnp.exp(l - m) for l in logits]
    se = exps[0]
    for p in range(1, P):
      se = se + exps[p]
    acc = (exps[0] / se) * emb_ref[0]
    for p in range(1, P):
      acc = acc + (exps[p] / se) * emb_ref[p]
    z_ref[...] = acc

  return pl.pallas_call(
      body,
      grid=(nb,),
      in_specs=[
          pl.BlockSpec((P, BN, H), lambda i: (0, i, 0)),
          pl.BlockSpec((P, H), lambda i: (0, 0)),
          pl.BlockSpec((1, H), lambda i: (0, 0)),
      ],
      out_specs=pl.BlockSpec((BN, H), lambda i: (i, 0)),
      out_shape=jax.ShapeDtypeStruct((N, H), jnp.float32),
  )(emb, sacc, att.reshape(1, H))


def kernel(h, edge_index, edge_weight, W, b, prelu_a, fc_W, fc_b, att):
  N, H = h.shape
  P = edge_index.shape[0]
  dst = edge_index[:, 0, :]
  src = edge_index[:, 1, :]
  parts = _sc_aggregate(h, src, dst, edge_weight)
  emb, sacc = _tc_transform(parts, W, b, prelu_a, fc_W, fc_b)
  return _tc_combine(emb, sacc, att, float(N))


# speedup vs baseline: 5.3093x; 1.2181x over previous
import functools

import jax
import jax.numpy as jnp
from jax import lax
from jax.experimental import pallas as pl
from jax.experimental.pallas import tpu as pltpu
from jax.experimental.pallas import tpu_sc as plsc

NC = 2
NS = 16
LN = 16


def _sc_aggregate(h, src, dst, w):
  N, H = h.shape
  P, E = src.shape
  src = src.reshape(P * E)
  dst = dst.reshape(P * E)
  w = w.reshape(P * E)
  NW = NC * NS
  per_w = E // NW
  CH = 80
  n_chunks = per_w // CH
  tail = per_w - n_chunks * CH
  RA = (N // NS) // 8 * 8
  RB = N - RA * (NS - 1)
  assert RB % 8 == 0 and RB <= 2 * RA
  mesh = plsc.VectorSubcoreMesh(core_axis_name="c", subcore_axis_name="s",
                                num_cores=NC, num_subcores=NS)

  def _scale_rows(rows_ref, w_ref, k):
    def body(g, carry):
      wv = w_ref[pl.ds(g * LN, LN)]
      for j in range(LN):
        wi = wv[j]
        row = g * LN + j
        for c in range(H // LN):
          sl = pl.ds(c * LN, LN)
          rows_ref[row, sl] = rows_ref[row, sl] * wi
      return carry
    lax.fori_loop(0, k // LN, body, 0)

  @functools.partial(
      pl.kernel,
      out_type=jax.ShapeDtypeStruct((P, NC, N, H), jnp.float32),
      mesh=mesh,
      scratch_types=[
          pltpu.VMEM_SHARED((N, H), jnp.float32),
          [pltpu.VMEM((CH,), jnp.int32) for _ in range(2)],
          [pltpu.VMEM((CH,), jnp.int32) for _ in range(2)],
          [pltpu.VMEM((CH,), jnp.float32) for _ in range(2)],
          pltpu.VMEM((2, CH, H), jnp.float32),
          pltpu.VMEM((LN,), jnp.int32),
          pltpu.VMEM((LN,), jnp.int32),
          pltpu.VMEM((LN,), jnp.float32),
          pltpu.VMEM((LN, H), jnp.float32),
          pltpu.SemaphoreType.DMA((2,)),
          pltpu.SemaphoreType.DMA((2,)),
      ],
  )
  def body(h_hbm, src_hbm, dst_hbm, w_hbm, zero_hbm, parts_hbm,
           acc, src_v, dst_v, w_v, rows_v, src_t, dst_t, w_t, rows_t,
           isem, gsem):
    cid = lax.axis_index("c")
    sid = lax.axis_index("s")
    wid = sid * NC + cid

    def idx_start(off, b):
      pltpu.async_copy(src_hbm.at[pl.ds(off, CH)], src_v[b], isem.at[b])
      pltpu.async_copy(dst_hbm.at[pl.ds(off, CH)], dst_v[b], isem.at[b])
      pltpu.async_copy(w_hbm.at[pl.ds(off, CH)], w_v[b], isem.at[b])

    def idx_wait(off, b):
      pltpu.make_async_copy(src_hbm.at[pl.ds(off, CH)], src_v[b],
                            isem.at[b]).wait()
      pltpu.make_async_copy(dst_hbm.at[pl.ds(off, CH)], dst_v[b],
                            isem.at[b]).wait()
      pltpu.make_async_copy(w_hbm.at[pl.ds(off, CH)], w_v[b],
                            isem.at[b]).wait()

    def gather_start(b):
      pltpu.async_copy(h_hbm.at[src_v[b]], rows_v.at[b], gsem.at[b])

    def gather_wait(b):
      pltpu.make_async_copy(h_hbm.at[src_v[b]], rows_v.at[b],
                            gsem.at[b]).wait()

    for p in range(P):
      base = p * E + wid * per_w
      @pl.when(sid < NS - 1)
      def _():
        pltpu.sync_copy(zero_hbm.at[pl.ds(0, RA)],
                        acc.at[pl.ds(sid * RA, RA)])

      @pl.when(sid == NS - 1)
      def _():
        pltpu.sync_copy(zero_hbm, acc.at[pl.ds((NS - 1) * RA, RB)])

      plsc.subcore_barrier()
      idx_start(base, 0)
      idx_wait(base, 0)
      gather_start(0)
      if n_chunks > 1:
        idx_start(base + CH, 1)

      def chunk_body(i, bb):
        off = base + i * CH
        gather_wait(bb)

        @pl.when(i + 1 < n_chunks)
        def _():
          idx_wait(off + CH, 1 - bb)
          gather_start(1 - bb)
        _scale_rows(rows_v.at[bb], w_v[bb], CH)
        pltpu.sync_copy(rows_v.at[bb], acc.at[dst_v[bb]], add=True)

        @pl.when(i + 2 < n_chunks)
        def _():
          idx_start(off + 2 * CH, bb)

      nc2 = n_chunks // 2 * 2

      def chunk(k, carry):
        for bb in range(2):
          chunk_body(k * 2 + bb, bb)
        return carry
      lax.fori_loop(0, nc2 // 2, chunk, 0)
      for i in range(nc2, n_chunks):
        chunk_body(i, i % 2)

      if tail:
        off = base + n_chunks * CH
        pltpu.sync_copy(src_hbm.at[pl.ds(off, tail)], src_t)
        pltpu.sync_copy(dst_hbm.at[pl.ds(off, tail)], dst_t)
        pltpu.sync_copy(w_hbm.at[pl.ds(off, tail)], w_t)
        pltpu.async_copy(h_hbm.at[src_t], rows_t, gsem.at[0]).wait()
        _scale_rows(rows_t, w_t, tail)
        pltpu.sync_copy(rows_t, acc.at[dst_t], add=True)

      plsc.subcore_barrier()

      @pl.when(sid < NS - 1)
      def _():
        pltpu.sync_copy(
            acc.at[pl.ds(sid * RA, RA)],
            parts_hbm.at[p, cid, pl.ds(sid * RA, RA)])

      @pl.when(sid == NS - 1)
      def _():
        pltpu.sync_copy(
            acc.at[pl.ds((NS - 1) * RA, RB)],
            parts_hbm.at[p, cid, pl.ds((NS - 1) * RA, RB)])

      plsc.subcore_barrier()

  zeros_slab = jnp.zeros((RB, H), dtype=jnp.float32)
  return body(h, src, dst, w, zeros_slab)


def _tc_transform(parts, W, b, prelu_a, fc_W, fc_b):
  P, _, N, H = parts.shape
  BN = 1000 if N % 1000 == 0 else N
  nb = N // BN

  def body(parts_ref, W_ref, b_ref, a_ref, fcW_ref, fcb_ref,
           emb_ref, sacc_ref):
    i = pl.program_id(0)

    @pl.when(i == 0)
    def _():
      sacc_ref[...] = jnp.zeros_like(sacc_ref)

    for p in range(P):
      agg = parts_ref[p, 0] + parts_ref[p, 1]
      fts = lax.dot_general(agg, W_ref[p], (((1,), (1,)), ((), ())),
                            preferred_element_type=jnp.float32)
      x = fts + b_ref[p:p + 1, :]
      a = a_ref[0, p]
      e = jnp.where(x > 0, x, a * x)
      emb_ref[p] = e
      t = jnp.tanh(
          lax.dot_general(e, fcW_ref[...], (((1,), (1,)), ((), ())),
                          preferred_element_type=jnp.float32)
          + fcb_ref[...])
      sacc_ref[p:p + 1, :] += jnp.sum(t, axis=0, keepdims=True)

  emb, sacc = pl.pallas_call(
      body,
      grid=(nb,),
      in_specs=[
          pl.BlockSpec((P, 2, BN, H), lambda i: (0, 0, i, 0)),
          pl.BlockSpec((P, H, H), lambda i: (0, 0, 0)),
          pl.BlockSpec((P, H), lambda i: (0, 0)),
          pl.BlockSpec(memory_space=pltpu.SMEM),
          pl.BlockSpec((H, H), lambda i: (0, 0)),
          pl.BlockSpec((1, H), lambda i: (0, 0)),
      ],
      out_specs=[
          pl.BlockSpec((P, BN, H), lambda i: (0, i, 0)),
          pl.BlockSpec((P, H), lambda i: (0, 0)),
      ],
      out_shape=[
          jax.ShapeDtypeStruct((P, N, H), jnp.float32),
          jax.ShapeDtypeStruct((P, H), jnp.float32),
      ],
  )(parts, W, b, prelu_a.reshape(1, P), fc_W, fc_b.reshape(1, H))
  return emb, sacc


def _tc_combine(emb, sacc, att, n_nodes):
  P, N, H = emb.shape
  BN = 1000 if N % 1000 == 0 else N
  nb = N // BN

  def body(emb_ref, sacc_ref, att_ref, z_ref):
    logits = [
        jnp.sum(att_ref[...] * sacc_ref[p:p + 1, :], axis=1, keepdims=True)
        / n_nodes
        for p in range(P)
    ]
    m = logits[0]
    for p in range(1, P):
      m = jnp.maximum(m, logits[p])
    exps = [jnp.exp(l - m) for l in logits]
    se = exps[0]
    for p in range(1, P):
      se = se + exps[p]
    acc = (exps[0] / se) * emb_ref[0]
    for p in range(1, P):
      acc = acc + (exps[p] / se) * emb_ref[p]
    z_ref[...] = acc

  return pl.pallas_call(
      body,
      grid=(nb,),
      in_specs=[
          pl.BlockSpec((P, BN, H), lambda i: (0, i, 0)),
          pl.BlockSpec((P, H), lambda i: (0, 0)),
          pl.BlockSpec((1, H), lambda i: (0, 0)),
      ],
      out_specs=pl.BlockSpec((BN, H), lambda i: (i, 0)),
      out_shape=jax.ShapeDtypeStruct((N, H), jnp.float32),
  )(emb, sacc, att.reshape(1, H))


def kernel(h, edge_index, edge_weight, W, b, prelu_a, fc_W, fc_b, att):
  N, H = h.shape
  P = edge_index.shape[0]
  dst = edge_index[:, 0, :]
  src = edge_index[:, 1, :]
  parts = _sc_aggregate(h, src, dst, edge_weight)
  emb, sacc = _tc_transform(parts, W, b, prelu_a, fc_W, fc_b)
  return _tc_combine(emb, sacc, att, float(N))
